# baseline (speedup 1.0000x reference)
import functools

import jax
import jax.numpy as jnp
from jax import lax
from jax.experimental import pallas as pl
from jax.experimental.pallas import tpu as pltpu
from jax.experimental.pallas import tpu_sc as plsc

_NC = 2
_NS = 16
_NW = _NC * _NS
_L = 16


def _build_emb(J, B0, D):
    n_cols = B0 // 128
    tc_per_w = n_cols // _NW
    mesh = plsc.VectorSubcoreMesh(core_axis_name="c", subcore_axis_name="s")

    @functools.partial(
        pl.kernel,
        out_type=jax.ShapeDtypeStruct((J, D // 8, n_cols, 8, 128), jnp.float32),
        mesh=mesh,
        scratch_types=[
            pltpu.VMEM((tc_per_w, 128), jnp.int32),
            pltpu.VMEM((2, 128, D), jnp.float32),
            pltpu.VMEM((2, D // 8, 8, 128), jnp.float32),
            pltpu.SemaphoreType.DMA((2,)),
            pltpu.SemaphoreType.DMA((2,)),
        ],
        compiler_params=pltpu.CompilerParams(
            use_tc_tiling_on_sc=False, needs_layout_passes=False),
    )
    def emb(xt_hbm, w_hbm, out_hbm, xbuf, gbuf, obuf, gsem, wsem):
        wid = lax.axis_index("s") * _NC + lax.axis_index("c")
        tc0 = wid * tc_per_w

        rows_g = [lax.iota(jnp.int32, _L) + jnp.int32(g * _L)
                  for g in range(128 // _L)]

        def gather(k, b):
            return pltpu.async_copy(
                w_hbm.at[xbuf.at[k]], gbuf.at[b], gsem.at[b])

        def extract(b):
            for c in range(D):
                cvec = jnp.full((_L,), c, jnp.int32)
                for g in range(128 // _L):
                    vals = plsc.load_gather(gbuf.at[b], [rows_g[g], cvec])
                    obuf[b, c // 8, c % 8, pl.ds(g * _L, _L)] = vals

        def writeout(j, k, b):
            return pltpu.async_copy(
                obuf.at[b], out_hbm.at[j, :, tc0 + k], wsem.at[b])

        def jbody(j, carry):
            pltpu.sync_copy(xt_hbm.at[j, pl.ds(tc0, tc_per_w)], xbuf)
            g = [gather(0, 0), gather(1, 1)]
            w_prev = [None, None]
            for k in range(tc_per_w):
                b = k % 2
                g[b].wait()
                if w_prev[b] is not None:
                    w_prev[b].wait()
                extract(b)
                w_prev[b] = writeout(j, k, b)
                if k + 2 < tc_per_w:
                    g[b] = gather(k + 2, b)
            w_prev[0].wait()
            w_prev[1].wait()
            return carry

        lax.fori_loop(0, J, jbody, jnp.int32(0))

    return emb


def kernel(x, w):
    B0, J = x.shape
    V, D = w.shape
    xt = x.T.astype(jnp.int32).reshape(J, B0 // 128, 128)
    out5 = _build_emb(J, B0, D)(xt, w)
    out_t = out5.transpose(0, 1, 3, 2, 4).reshape(J, D, B0)
    return out_t.transpose(2, 0, 1)


# speedup vs baseline: 1.2543x; 1.2543x over previous
import functools

import jax
import jax.numpy as jnp
from jax import lax
from jax.experimental import pallas as pl
from jax.experimental.pallas import tpu as pltpu
from jax.experimental.pallas import tpu_sc as plsc

_NC = 2
_NS = 16
_NW = _NC * _NS
_L = 16


def _build_emb(J, B0, D):
    n_cols = B0 // 128
    tc_per_w = n_cols // _NW
    mesh = plsc.VectorSubcoreMesh(core_axis_name="c", subcore_axis_name="s")

    @functools.partial(
        pl.kernel,
        out_type=jax.ShapeDtypeStruct((J, D // 8, n_cols, 8, 128), jnp.float32),
        mesh=mesh,
        scratch_types=[
            pltpu.VMEM((J, tc_per_w, 128), jnp.int32),
            pltpu.VMEM((4, 128, D), jnp.float32),
            pltpu.VMEM((2, D // 8, 8, 128), jnp.float32),
            pltpu.SemaphoreType.DMA((4,)),
            pltpu.SemaphoreType.DMA((2,)),
        ],
        compiler_params=pltpu.CompilerParams(
            use_tc_tiling_on_sc=False, needs_layout_passes=False),
    )
    def emb(xt_hbm, w_hbm, out_hbm, xbuf, gbuf, obuf, gsem, wsem):
        wid = lax.axis_index("s") * _NC + lax.axis_index("c")
        tc0 = wid * tc_per_w

        rows_g = [lax.iota(jnp.int32, _L) + jnp.int32(g * _L)
                  for g in range(128 // _L)]

        def gather(j, k):
            return pltpu.async_copy(
                w_hbm.at[xbuf.at[j, k]], gbuf.at[k], gsem.at[k])

        cvecs = [jnp.full((_L,), c, jnp.int32) for c in range(D)]

        def extract(k, ob):
            for g in range(128 // _L):
                for c0 in range(0, D, 8):
                    vals = [
                        plsc.load_gather(gbuf.at[k], [rows_g[g], cvecs[c0 + d]])
                        for d in range(8)
                    ]
                    for d in range(8):
                        c = c0 + d
                        obuf[ob, c // 8, c % 8, pl.ds(g * _L, _L)] = vals[d]

        def writeout(j, k, ob):
            return pltpu.async_copy(
                obuf.at[ob], out_hbm.at[j, :, tc0 + k], wsem.at[ob])

        pltpu.sync_copy(xt_hbm.at[:, pl.ds(tc0, tc_per_w)], xbuf)

        def jbody(j, carry):
            g = [gather(j, k) for k in range(tc_per_w)]
            w_prev = [None, None]
            for k in range(tc_per_w):
                ob = k % 2
                g[k].wait()
                if w_prev[ob] is not None:
                    w_prev[ob].wait()
                extract(k, ob)
                w_prev[ob] = writeout(j, k, ob)
            w_prev[0].wait()
            w_prev[1].wait()
            return carry

        lax.fori_loop(0, J, jbody, jnp.int32(0))

    return emb


def kernel(x, w):
    B0, J = x.shape
    V, D = w.shape
    xt = x.T.astype(jnp.int32).reshape(J, B0 // 128, 128)
    out5 = _build_emb(J, B0, D)(xt, w)
    out_t = out5.transpose(0, 1, 3, 2, 4).reshape(J, D, B0)
    return out_t.transpose(2, 0, 1)


# speedup vs baseline: 1.3052x; 1.0406x over previous
import functools

import jax
import jax.numpy as jnp
from jax import lax
from jax.experimental import pallas as pl
from jax.experimental.pallas import tpu as pltpu
from jax.experimental.pallas import tpu_sc as plsc

_NC = 2
_NS = 16
_NW = _NC * _NS
_L = 16


def _build_emb(J, B0, D):
    n_cols = B0 // 128
    tc_per_w = n_cols // _NW
    mesh = plsc.VectorSubcoreMesh(core_axis_name="c", subcore_axis_name="s")

    @functools.partial(
        pl.kernel,
        out_type=jax.ShapeDtypeStruct((J, D // 8, n_cols, 8, 128), jnp.float32),
        mesh=mesh,
        scratch_types=[
            pltpu.VMEM((J, tc_per_w, 128), jnp.int32),
            pltpu.VMEM((4, 128, D), jnp.float32),
            pltpu.VMEM((2, D // 8, 8, 128), jnp.float32),
            pltpu.SemaphoreType.DMA((4,)),
            pltpu.SemaphoreType.DMA((2,)),
        ],
        compiler_params=pltpu.CompilerParams(
            use_tc_tiling_on_sc=False, needs_layout_passes=False),
    )
    def emb(xt_hbm, w_hbm, out_hbm, xbuf, gbuf, obuf, gsem, wsem):
        wid = lax.axis_index("s") * _NC + lax.axis_index("c")
        tc0 = wid * tc_per_w

        rows_g = [lax.iota(jnp.int32, _L) + jnp.int32(g * _L)
                  for g in range(128 // _L)]

        def gather(j, k):
            return pltpu.async_copy(
                w_hbm.at[xbuf.at[j, k]], gbuf.at[k], gsem.at[k])

        cvecs = [jnp.full((_L,), c, jnp.int32) for c in range(D)]

        def extract(k, ob):
            for g in range(128 // _L):
                for c0 in range(0, D, 8):
                    vals = [
                        plsc.load_gather(gbuf.at[k], [rows_g[g], cvecs[c0 + d]])
                        for d in range(8)
                    ]
                    for d in range(8):
                        c = c0 + d
                        obuf[ob, c // 8, c % 8, pl.ds(g * _L, _L)] = vals[d]

        def writeout(j, k, ob):
            return pltpu.async_copy(
                obuf.at[ob], out_hbm.at[j, :, tc0 + k], wsem.at[ob])

        def wait_gather(k):
            pltpu.make_async_copy(
                w_hbm.at[xbuf.at[0, 0]], gbuf.at[k], gsem.at[k]).wait()

        pltpu.sync_copy(xt_hbm.at[:, pl.ds(tc0, tc_per_w)], xbuf)

        for k in range(tc_per_w):
            gather(0, k)

        def jbody(j, carry):
            jn = jnp.minimum(j + 1, J - 1)
            w_prev = [None, None]
            for k in range(tc_per_w):
                ob = k % 2
                wait_gather(k)
                if w_prev[ob] is not None:
                    w_prev[ob].wait()
                extract(k, ob)
                w_prev[ob] = writeout(j, k, ob)
                gather(jn, k)
            w_prev[0].wait()
            w_prev[1].wait()
            return carry

        lax.fori_loop(0, J, jbody, jnp.int32(0))
        for k in range(tc_per_w):
            wait_gather(k)

    return emb


def kernel(x, w):
    B0, J = x.shape
    V, D = w.shape
    xt = x.T.astype(jnp.int32).reshape(J, B0 // 128, 128)
    out5 = _build_emb(J, B0, D)(xt, w)
    out_t = out5.transpose(0, 1, 3, 2, 4).reshape(J, D, B0)
    return out_t.transpose(2, 0, 1)
